# Initial kernel scaffold; baseline (speedup 1.0000x reference)
import jax
import jax.numpy as jnp
from jax.experimental import pallas as pl


def kernel(x, edge_index, edge_type, W_des, b_des, W_num, b_num, W_cat, b_cat, W_tot, b_tot, a_emb, W1, b1, W2, b2, Wr, br, Wp1, bp1, Wp2, bp2):
    raise NotImplementedError("write your pallas kernel here")


# speedup vs baseline: 8.7394x; 8.7394x over previous
import functools

import jax
import jax.numpy as jnp
from jax import lax
from jax.experimental import pallas as pl
from jax.experimental.pallas import tpu as pltpu
from jax.experimental.pallas import tpu_sc as plsc

N_NODES = 10000
HID = 128
NC = 2
NS = 16
LANES = 128
IDX_ROWS = 8
EDGE_BLK = IDX_ROWS * LANES
ACC_ROWS = 10240
ROWS_PER_TILE = ACC_ROWS // NS
DEG_W = 16


def _prelu(x, a):
    return jnp.where(x >= 0, x, a * x)


DEG_PANEL = ACC_ROWS // 8


def _seg_body(nblk, table, src_i, dst_i, z128,
              out, deg_out, acc, dacc, srcv, dstv, rowsv, ones1, dstage, sem):
    c = lax.axis_index("c")
    s = lax.axis_index("s")
    row0 = s * ROWS_PER_TILE
    pltpu.sync_copy(z128, rowsv)
    zero16 = jnp.zeros((16,), jnp.float32)
    for i in range(ROWS_PER_TILE // 16):
        dstage[pl.ds(i * 16, 16)] = zero16
    for t in range(ROWS_PER_TILE // LANES):
        pltpu.sync_copy(rowsv, acc.at[pl.ds(row0 + t * LANES, LANES)])
    pltpu.sync_copy(dstage, dacc.at[pl.ds(row0, ROWS_PER_TILE)])
    one16 = jnp.ones((16,), jnp.float32)
    for i in range(LANES // 16):
        ones1[pl.ds(i * 16, 16)] = one16
    plsc.subcore_barrier()

    idx_row_base = (c * NS + s) * (nblk * IDX_ROWS)

    @pl.loop(0, nblk)
    def _outer(b):
        r = idx_row_base + b * IDX_ROWS
        pltpu.sync_copy(src_i.at[pl.ds(r, IDX_ROWS)], srcv)
        pltpu.sync_copy(dst_i.at[pl.ds(r, IDX_ROWS)], dstv)

        for j in range(IDX_ROWS):
            pltpu.async_copy(table.at[srcv.at[j]], rowsv, sem).wait()
            pltpu.sync_copy(rowsv, acc.at[dstv.at[j]], add=True)
            pltpu.sync_copy(ones1, dacc.at[dstv.at[j]], add=True)

    plsc.subcore_barrier()
    base = c * ACC_ROWS + row0
    for t in range(ROWS_PER_TILE // LANES):
        pltpu.sync_copy(acc.at[pl.ds(row0 + t * LANES, LANES)], rowsv)
        pltpu.sync_copy(rowsv, out.at[pl.ds(base + t * LANES, LANES)])
    pltpu.sync_copy(dacc.at[pl.ds(row0, ROWS_PER_TILE)], dstage)
    pltpu.sync_copy(dstage, deg_out.at[pl.ds(base, ROWS_PER_TILE)])


def _make_seg_kernel(n_edges_pad):
    nblk = n_edges_pad // (NC * NS * EDGE_BLK)
    mesh = plsc.VectorSubcoreMesh(core_axis_name="c", subcore_axis_name="s")
    return pl.kernel(
        functools.partial(_seg_body, nblk),
        out_type=(
            jax.ShapeDtypeStruct((NC * ACC_ROWS, HID), jnp.float32),
            jax.ShapeDtypeStruct((NC * ACC_ROWS,), jnp.float32),
        ),
        mesh=mesh,
        scratch_types=[
            pltpu.VMEM_SHARED((ACC_ROWS, HID), jnp.float32),
            pltpu.VMEM_SHARED((ACC_ROWS,), jnp.float32),
            pltpu.VMEM((IDX_ROWS, LANES), jnp.int32),
            pltpu.VMEM((IDX_ROWS, LANES), jnp.int32),
            pltpu.VMEM((LANES, HID), jnp.float32),
            pltpu.VMEM((LANES,), jnp.float32),
            pltpu.VMEM((ROWS_PER_TILE,), jnp.float32),
            pltpu.SemaphoreType.DMA,
        ],
    )


def _stage_a_body(x, wf, ba, wt, bt, w1, b1, wr, br, a, g1, r):
    av = a[0, 0]
    e = _prelu(jnp.dot(x[...], wf[...], preferred_element_type=jnp.float32)
               + ba[...], av)
    h0 = _prelu(jnp.dot(e, wt[...], preferred_element_type=jnp.float32)
                + bt[...], av)
    g1[...] = jnp.dot(h0, w1[...], preferred_element_type=jnp.float32) + b1[...]
    r[...] = jnp.dot(h0, wr[...], preferred_element_type=jnp.float32) + br[...]


def _stage_b_body(sp, dp, w2, b2, g2):
    ssum = sp[0] + sp[1]
    deg = jnp.maximum(dp[0] + dp[1], 1.0)
    h = jnp.maximum(ssum / deg, 0.0)
    g2[...] = jnp.dot(h, w2[...], preferred_element_type=jnp.float32) + b2[...]


def _stage_c_body(sp, dp, r, wp1, bp1, wp2, bp2, out):
    ssum = sp[0] + sp[1]
    deg = jnp.maximum(dp[0] + dp[1], 1.0)
    h2 = jnp.maximum(ssum / deg, 0.0)
    t = h2 + r[...]
    p = jnp.maximum(jnp.dot(t, wp1[...], preferred_element_type=jnp.float32)
                    + bp1[...], 0.0)
    out[...] = jnp.dot(p, wp2[...], preferred_element_type=jnp.float32) + bp2[...]


def _full(shape):
    return pl.BlockSpec(shape, lambda i: tuple(0 for _ in shape))


def kernel(x, edge_index, edge_type, W_des, b_des, W_num, b_num, W_cat, b_cat,
           W_tot, b_tot, a_emb, W1, b1, W2, b2, Wr, br, Wp1, bp1, Wp2, bp2):
    del edge_type
    f32 = jnp.float32
    n = x.shape[0]
    k = x.shape[1]

    num_idx = jnp.array([4, 6, 7, 8, 10, 11, 12, 13, 14, 15], dtype=jnp.int32)
    cat_idx = jnp.array([1, 2, 3, 5, 9, 16, 17, 18, 19, 20], dtype=jnp.int32)
    wf = jnp.zeros((k, HID), f32)
    wf = wf.at[k - 768:, 0:96].set(W_des)
    wf = wf.at[num_idx, 96:112].set(W_num)
    wf = wf.at[cat_idx, 112:128].set(W_cat)
    ba = jnp.concatenate([b_des, b_num, b_cat]).reshape(1, HID)

    bm = 1000
    grid = (n // bm,)
    row_spec = pl.BlockSpec((bm, HID), lambda i: (i, 0))
    g1, r = pl.pallas_call(
        _stage_a_body,
        grid=grid,
        in_specs=[pl.BlockSpec((bm, k), lambda i: (i, 0)),
                  _full((k, HID)), _full((1, HID)),
                  _full((HID, HID)), _full((1, HID)),
                  _full((HID, HID)), _full((1, HID)),
                  _full((HID, HID)), _full((1, HID)),
                  _full((1, 1))],
        out_specs=[row_spec, row_spec],
        out_shape=[jax.ShapeDtypeStruct((n, HID), f32),
                   jax.ShapeDtypeStruct((n, HID), f32)],
    )(x, wf, ba, W_tot, b_tot.reshape(1, HID), W1, b1.reshape(1, HID),
      Wr, br.reshape(1, HID), a_emb.reshape(1, 1))

    src = edge_index[0]
    dst = edge_index[1]
    n_edges = src.shape[0]
    epad = ((n_edges + NC * NS * EDGE_BLK - 1) // (NC * NS * EDGE_BLK)
            * (NC * NS * EDGE_BLK))
    npad = epad - n_edges
    pad_i = jnp.arange(npad, dtype=jnp.int32)
    src_p = jnp.concatenate([src, pad_i % N_NODES]).reshape(-1, LANES)
    dst_p = jnp.concatenate(
        [dst, N_NODES + pad_i % (ACC_ROWS - N_NODES)]).reshape(-1, LANES)

    z128 = jnp.zeros((LANES, HID), f32)

    seg_k = _make_seg_kernel(epad)

    def seg(table):
        sums, deg = seg_k(table, src_p, dst_p, z128)
        return (sums.reshape(NC, ACC_ROWS, HID),
                deg.reshape(NC, ACC_ROWS, 1))

    s1, deg1 = seg(g1)

    bm2 = 1024
    grid2 = (ACC_ROWS // bm2,)
    g2 = pl.pallas_call(
        _stage_b_body,
        grid=grid2,
        in_specs=[pl.BlockSpec((NC, bm2, HID), lambda i: (0, i, 0)),
                  pl.BlockSpec((NC, bm2, 1), lambda i: (0, i, 0)),
                  _full((HID, HID)), _full((1, HID))],
        out_specs=pl.BlockSpec((bm2, HID), lambda i: (i, 0)),
        out_shape=jax.ShapeDtypeStruct((ACC_ROWS, HID), f32),
    )(s1, deg1, W2, b2.reshape(1, HID))

    s2, _ = seg(g2)

    wp2 = jnp.zeros((HID, HID), f32).at[:, :2].set(Wp2)
    bp2p = jnp.zeros((1, HID), f32).at[0, :2].set(bp2)
    out = pl.pallas_call(
        _stage_c_body,
        grid=grid,
        in_specs=[pl.BlockSpec((NC, bm, HID), lambda i: (0, i, 0)),
                  pl.BlockSpec((NC, bm, 1), lambda i: (0, i, 0)),
                  row_spec,
                  _full((HID, HID)), _full((1, HID)),
                  _full((HID, HID)), _full((1, HID))],
        out_specs=row_spec,
        out_shape=jax.ShapeDtypeStruct((n, HID), f32),
    )(s2, deg1, r, Wp1, bp1.reshape(1, HID), wp2, bp2p)

    return out[:, :2]


# speedup vs baseline: 9.6482x; 1.1040x over previous
import functools

import jax
import jax.numpy as jnp
from jax import lax
from jax.experimental import pallas as pl
from jax.experimental.pallas import tpu as pltpu
from jax.experimental.pallas import tpu_sc as plsc

N_NODES = 10000
HID = 128
NC = 2
NS = 16
LANES = 128
CHUNK = 64
IDX_ROWS = 8
EDGE_BLK = IDX_ROWS * CHUNK
ACC_ROWS = 10240
ROWS_PER_TILE = ACC_ROWS // NS


def _prelu(x, a):
    return jnp.where(x >= 0, x, a * x)


DEG_PANEL = ACC_ROWS // 8


def _seg_body(nblk, with_deg, table, src_i, dst_i, z128, *refs):
    if with_deg:
        (out, deg_out, acc, dacc, srcv, dstv, rowsa, rowsb,
         ones1, dstage, sema, semb) = refs
    else:
        out, acc, srcv, dstv, rowsa, rowsb, sema, semb = refs
    c = lax.axis_index("c")
    s = lax.axis_index("s")
    row0 = s * ROWS_PER_TILE
    pltpu.sync_copy(z128, rowsa)
    for t in range(ROWS_PER_TILE // CHUNK):
        pltpu.sync_copy(rowsa, acc.at[pl.ds(row0 + t * CHUNK, CHUNK)])
    if with_deg:
        zero16 = jnp.zeros((16,), jnp.float32)
        for i in range(ROWS_PER_TILE // 16):
            dstage[pl.ds(i * 16, 16)] = zero16
        pltpu.sync_copy(dstage, dacc.at[pl.ds(row0, ROWS_PER_TILE)])
        one16 = jnp.ones((16,), jnp.float32)
        for i in range(CHUNK // 16):
            ones1[pl.ds(i * 16, 16)] = one16
    plsc.subcore_barrier()

    idx_row_base = (c * NS + s) * (nblk * IDX_ROWS)
    bufs = (rowsa, rowsb)
    sems = (sema, semb)

    @pl.loop(0, nblk)
    def _outer(b):
        r = idx_row_base + b * IDX_ROWS
        pltpu.sync_copy(src_i.at[pl.ds(r, IDX_ROWS)], srcv)
        pltpu.sync_copy(dst_i.at[pl.ds(r, IDX_ROWS)], dstv)

        descs = [None] * IDX_ROWS
        descs[0] = pltpu.async_copy(table.at[srcv.at[0]], bufs[0], sems[0])
        for j in range(IDX_ROWS):
            if j + 1 < IDX_ROWS:
                descs[j + 1] = pltpu.async_copy(
                    table.at[srcv.at[j + 1]], bufs[(j + 1) % 2],
                    sems[(j + 1) % 2])
            descs[j].wait()
            pltpu.sync_copy(bufs[j % 2], acc.at[dstv.at[j]], add=True)
            if with_deg:
                pltpu.sync_copy(ones1, dacc.at[dstv.at[j]], add=True)

    plsc.subcore_barrier()
    base = c * ACC_ROWS + row0
    for t in range(ROWS_PER_TILE // CHUNK):
        pltpu.sync_copy(acc.at[pl.ds(row0 + t * CHUNK, CHUNK)], rowsa)
        pltpu.sync_copy(rowsa, out.at[pl.ds(base + t * CHUNK, CHUNK)])
    if with_deg:
        pltpu.sync_copy(dacc.at[pl.ds(row0, ROWS_PER_TILE)], dstage)
        pltpu.sync_copy(dstage, deg_out.at[pl.ds(base, ROWS_PER_TILE)])


def _make_seg_kernel(n_edges_pad, with_deg):
    nblk = n_edges_pad // (NC * NS * EDGE_BLK)
    mesh = plsc.VectorSubcoreMesh(core_axis_name="c", subcore_axis_name="s")
    outs = [jax.ShapeDtypeStruct((NC * ACC_ROWS, HID), jnp.float32)]
    scratch = [
        pltpu.VMEM_SHARED((ACC_ROWS, HID), jnp.float32),
    ]
    if with_deg:
        outs.append(jax.ShapeDtypeStruct((NC * ACC_ROWS,), jnp.float32))
        scratch.append(pltpu.VMEM_SHARED((ACC_ROWS,), jnp.float32))
    scratch += [
        pltpu.VMEM((IDX_ROWS, CHUNK), jnp.int32),
        pltpu.VMEM((IDX_ROWS, CHUNK), jnp.int32),
        pltpu.VMEM((CHUNK, HID), jnp.float32),
        pltpu.VMEM((CHUNK, HID), jnp.float32),
    ]
    if with_deg:
        scratch += [
            pltpu.VMEM((CHUNK,), jnp.float32),
            pltpu.VMEM((ROWS_PER_TILE,), jnp.float32),
        ]
    scratch += [pltpu.SemaphoreType.DMA, pltpu.SemaphoreType.DMA]
    return pl.kernel(
        functools.partial(_seg_body, nblk, with_deg),
        out_type=tuple(outs) if with_deg else outs[0],
        mesh=mesh,
        scratch_types=scratch,
    )


def _stage_a_body(x, wf, ba, wt, bt, w1, b1, wr, br, a, g1, r):
    av = a[0, 0]
    e = _prelu(jnp.dot(x[...], wf[...], preferred_element_type=jnp.float32)
               + ba[...], av)
    h0 = _prelu(jnp.dot(e, wt[...], preferred_element_type=jnp.float32)
                + bt[...], av)
    g1[...] = jnp.dot(h0, w1[...], preferred_element_type=jnp.float32) + b1[...]
    r[...] = jnp.dot(h0, wr[...], preferred_element_type=jnp.float32) + br[...]


def _stage_b_body(sp, dp, w2, b2, g2):
    ssum = sp[0] + sp[1]
    deg = jnp.maximum(dp[0] + dp[1], 1.0)
    h = jnp.maximum(ssum / deg, 0.0)
    g2[...] = jnp.dot(h, w2[...], preferred_element_type=jnp.float32) + b2[...]


def _stage_c_body(sp, dp, r, wp1, bp1, wp2, bp2, out):
    ssum = sp[0] + sp[1]
    deg = jnp.maximum(dp[0] + dp[1], 1.0)
    h2 = jnp.maximum(ssum / deg, 0.0)
    t = h2 + r[...]
    p = jnp.maximum(jnp.dot(t, wp1[...], preferred_element_type=jnp.float32)
                    + bp1[...], 0.0)
    out[...] = jnp.dot(p, wp2[...], preferred_element_type=jnp.float32) + bp2[...]


def _full(shape):
    return pl.BlockSpec(shape, lambda i: tuple(0 for _ in shape))


def kernel(x, edge_index, edge_type, W_des, b_des, W_num, b_num, W_cat, b_cat,
           W_tot, b_tot, a_emb, W1, b1, W2, b2, Wr, br, Wp1, bp1, Wp2, bp2):
    del edge_type
    f32 = jnp.float32
    n = x.shape[0]
    k = x.shape[1]

    num_idx = jnp.array([4, 6, 7, 8, 10, 11, 12, 13, 14, 15], dtype=jnp.int32)
    cat_idx = jnp.array([1, 2, 3, 5, 9, 16, 17, 18, 19, 20], dtype=jnp.int32)
    wf = jnp.zeros((k, HID), f32)
    wf = wf.at[k - 768:, 0:96].set(W_des)
    wf = wf.at[num_idx, 96:112].set(W_num)
    wf = wf.at[cat_idx, 112:128].set(W_cat)
    ba = jnp.concatenate([b_des, b_num, b_cat]).reshape(1, HID)

    bm = 1000
    grid = (n // bm,)
    row_spec = pl.BlockSpec((bm, HID), lambda i: (i, 0))
    g1, r = pl.pallas_call(
        _stage_a_body,
        grid=grid,
        in_specs=[pl.BlockSpec((bm, k), lambda i: (i, 0)),
                  _full((k, HID)), _full((1, HID)),
                  _full((HID, HID)), _full((1, HID)),
                  _full((HID, HID)), _full((1, HID)),
                  _full((HID, HID)), _full((1, HID)),
                  _full((1, 1))],
        out_specs=[row_spec, row_spec],
        out_shape=[jax.ShapeDtypeStruct((n, HID), f32),
                   jax.ShapeDtypeStruct((n, HID), f32)],
    )(x, wf, ba, W_tot, b_tot.reshape(1, HID), W1, b1.reshape(1, HID),
      Wr, br.reshape(1, HID), a_emb.reshape(1, 1))

    src = edge_index[0]
    dst = edge_index[1]
    n_edges = src.shape[0]
    epad = ((n_edges + NC * NS * EDGE_BLK - 1) // (NC * NS * EDGE_BLK)
            * (NC * NS * EDGE_BLK))
    npad = epad - n_edges
    pad_i = jnp.arange(npad, dtype=jnp.int32)
    src_p = jnp.concatenate([src, pad_i % N_NODES]).reshape(-1, CHUNK)
    dst_p = jnp.concatenate(
        [dst, N_NODES + pad_i % (ACC_ROWS - N_NODES)]).reshape(-1, CHUNK)

    z128 = jnp.zeros((CHUNK, HID), f32)

    s1, deg1 = _make_seg_kernel(epad, True)(g1, src_p, dst_p, z128)
    s1 = s1.reshape(NC, ACC_ROWS, HID)
    deg1 = deg1.reshape(NC, ACC_ROWS, 1)

    bm2 = 1024
    grid2 = (ACC_ROWS // bm2,)
    g2 = pl.pallas_call(
        _stage_b_body,
        grid=grid2,
        in_specs=[pl.BlockSpec((NC, bm2, HID), lambda i: (0, i, 0)),
                  pl.BlockSpec((NC, bm2, 1), lambda i: (0, i, 0)),
                  _full((HID, HID)), _full((1, HID))],
        out_specs=pl.BlockSpec((bm2, HID), lambda i: (i, 0)),
        out_shape=jax.ShapeDtypeStruct((ACC_ROWS, HID), f32),
    )(s1, deg1, W2, b2.reshape(1, HID))

    s2 = _make_seg_kernel(epad, False)(g2, src_p, dst_p, z128)
    s2 = s2.reshape(NC, ACC_ROWS, HID)

    wp2 = jnp.zeros((HID, HID), f32).at[:, :2].set(Wp2)
    bp2p = jnp.zeros((1, HID), f32).at[0, :2].set(bp2)
    out = pl.pallas_call(
        _stage_c_body,
        grid=grid,
        in_specs=[pl.BlockSpec((NC, bm, HID), lambda i: (0, i, 0)),
                  pl.BlockSpec((NC, bm, 1), lambda i: (0, i, 0)),
                  row_spec,
                  _full((HID, HID)), _full((1, HID)),
                  _full((HID, HID)), _full((1, HID))],
        out_specs=row_spec,
        out_shape=jax.ShapeDtypeStruct((n, HID), f32),
    )(s2, deg1, r, Wp1, bp1.reshape(1, HID), wp2, bp2p)

    return out[:, :2]


# speedup vs baseline: 10.6381x; 1.1026x over previous
import functools

import jax
import jax.numpy as jnp
from jax import lax
from jax.experimental import pallas as pl
from jax.experimental.pallas import tpu as pltpu
from jax.experimental.pallas import tpu_sc as plsc

N_NODES = 10000
HID = 128
NC = 2
NS = 16
LANES = 128
CHUNK = 64
IDX_ROWS = 8
EDGE_BLK = IDX_ROWS * CHUNK
ACC_ROWS = 10240
ROWS_PER_TILE = ACC_ROWS // NS


def _prelu(x, a):
    return jnp.where(x >= 0, x, a * x)


DEG_PANEL = ACC_ROWS // 8


def _seg_body(nblk, with_deg, table, src_i, dst_i, z128, *refs):
    if with_deg:
        (out, deg_out, acc, dacc, srcv, dstv, rowsa, rowsb,
         ones1, dstage, sema, semb, semc, semd) = refs
    else:
        out, acc, srcv, dstv, rowsa, rowsb, sema, semb, semc, semd = refs
    c = lax.axis_index("c")
    s = lax.axis_index("s")
    row0 = s * ROWS_PER_TILE
    nchunk = ROWS_PER_TILE // CHUNK
    pltpu.sync_copy(z128, rowsa)
    zdescs = [pltpu.async_copy(rowsa, acc.at[pl.ds(row0 + t * CHUNK, CHUNK)],
                               semc)
              for t in range(nchunk)]
    if with_deg:
        zero16 = jnp.zeros((16,), jnp.float32)
        for i in range(ROWS_PER_TILE // 16):
            dstage[pl.ds(i * 16, 16)] = zero16
        pltpu.sync_copy(dstage, dacc.at[pl.ds(row0, ROWS_PER_TILE)])
        one16 = jnp.ones((16,), jnp.float32)
        for i in range(CHUNK // 16):
            ones1[pl.ds(i * 16, 16)] = one16
    for d in zdescs:
        d.wait()
    plsc.subcore_barrier()

    idx_row_base = (c * NS + s) * (nblk * IDX_ROWS)
    bufs = (rowsa, rowsb)
    sems = (sema, semb)
    npair = nblk // 2

    def idx_load(blk, p):
        r = idx_row_base + blk * IDX_ROWS
        return (pltpu.async_copy(src_i.at[pl.ds(r, IDX_ROWS)], srcv.at[p],
                                 semc),
                pltpu.async_copy(dst_i.at[pl.ds(r, IDX_ROWS)], dstv.at[p],
                                 semd))

    idx_load(0, 0)

    def process(p):
        descs = [None] * IDX_ROWS
        descs[0] = pltpu.async_copy(table.at[srcv.at[p, 0]], bufs[0], sems[0])
        for j in range(IDX_ROWS):
            if j + 1 < IDX_ROWS:
                descs[j + 1] = pltpu.async_copy(
                    table.at[srcv.at[p, j + 1]], bufs[(j + 1) % 2],
                    sems[(j + 1) % 2])
            descs[j].wait()
            pltpu.sync_copy(bufs[j % 2], acc.at[dstv.at[p, j]], add=True)
            if with_deg:
                pltpu.sync_copy(ones1, dacc.at[dstv.at[p, j]], add=True)

    def idx_drain(p):
        pltpu.make_async_copy(src_i.at[pl.ds(0, IDX_ROWS)], srcv.at[p],
                              semc).wait()
        pltpu.make_async_copy(dst_i.at[pl.ds(0, IDX_ROWS)], dstv.at[p],
                              semd).wait()

    @pl.loop(0, npair)
    def _outer(bb):
        b0 = bb * 2
        idx_load(b0 + 1, 1)
        idx_drain(0)
        process(0)
        idx_load(b0 + 2, 0)
        idx_drain(1)
        process(1)

    idx_drain(0)
    plsc.subcore_barrier()
    base = c * ACC_ROWS + row0
    loads = [None] * nchunk
    outs = [None] * nchunk
    loads[0] = pltpu.async_copy(acc.at[pl.ds(row0, CHUNK)], rowsa, sema)
    for t in range(nchunk):
        bt = bufs[t % 2]
        if t + 1 < nchunk:
            if t >= 1:
                outs[t - 1].wait()
            loads[t + 1] = pltpu.async_copy(
                acc.at[pl.ds(row0 + (t + 1) * CHUNK, CHUNK)],
                bufs[(t + 1) % 2], sems[(t + 1) % 2])
        loads[t].wait()
        outs[t] = pltpu.async_copy(bt, out.at[pl.ds(base + t * CHUNK, CHUNK)],
                                   semc if t % 2 == 0 else semd)
    outs[nchunk - 2].wait()
    outs[nchunk - 1].wait()
    if with_deg:
        pltpu.sync_copy(dacc.at[pl.ds(row0, ROWS_PER_TILE)], dstage)
        pltpu.sync_copy(dstage, deg_out.at[pl.ds(base, ROWS_PER_TILE)])


def _make_seg_kernel(n_edges_pad, with_deg):
    nblk = n_edges_pad // (NC * NS * EDGE_BLK)
    mesh = plsc.VectorSubcoreMesh(core_axis_name="c", subcore_axis_name="s")
    outs = [jax.ShapeDtypeStruct((NC * ACC_ROWS, HID), jnp.float32)]
    scratch = [
        pltpu.VMEM_SHARED((ACC_ROWS, HID), jnp.float32),
    ]
    if with_deg:
        outs.append(jax.ShapeDtypeStruct((NC * ACC_ROWS,), jnp.float32))
        scratch.append(pltpu.VMEM_SHARED((ACC_ROWS,), jnp.float32))
    scratch += [
        pltpu.VMEM((2, IDX_ROWS, CHUNK), jnp.int32),
        pltpu.VMEM((2, IDX_ROWS, CHUNK), jnp.int32),
        pltpu.VMEM((CHUNK, HID), jnp.float32),
        pltpu.VMEM((CHUNK, HID), jnp.float32),
    ]
    if with_deg:
        scratch += [
            pltpu.VMEM((CHUNK,), jnp.float32),
            pltpu.VMEM((ROWS_PER_TILE,), jnp.float32),
        ]
    scratch += [pltpu.SemaphoreType.DMA] * 4
    return pl.kernel(
        functools.partial(_seg_body, nblk, with_deg),
        out_type=tuple(outs) if with_deg else outs[0],
        mesh=mesh,
        scratch_types=scratch,
    )


def _stage_a_body(x, wf, ba, wt, bt, w1, b1, wr, br, a, g1, r):
    av = a[0, 0]
    e = _prelu(jnp.dot(x[...], wf[...], preferred_element_type=jnp.float32)
               + ba[...], av)
    h0 = _prelu(jnp.dot(e, wt[...], preferred_element_type=jnp.float32)
                + bt[...], av)
    g1[...] = jnp.dot(h0, w1[...], preferred_element_type=jnp.float32) + b1[...]
    r[...] = jnp.dot(h0, wr[...], preferred_element_type=jnp.float32) + br[...]


def _stage_b_body(sp, dp, w2, b2, g2):
    ssum = sp[0] + sp[1]
    deg = jnp.maximum(dp[0] + dp[1], 1.0)
    h = jnp.maximum(ssum / deg, 0.0)
    g2[...] = jnp.dot(h, w2[...], preferred_element_type=jnp.float32) + b2[...]


def _stage_c_body(sp, dp, r, wp1, bp1, wp2, bp2, out):
    ssum = sp[0] + sp[1]
    deg = jnp.maximum(dp[0] + dp[1], 1.0)
    h2 = jnp.maximum(ssum / deg, 0.0)
    t = h2 + r[...]
    p = jnp.maximum(jnp.dot(t, wp1[...], preferred_element_type=jnp.float32)
                    + bp1[...], 0.0)
    out[...] = jnp.dot(p, wp2[...], preferred_element_type=jnp.float32) + bp2[...]


def _full(shape):
    return pl.BlockSpec(shape, lambda i: tuple(0 for _ in shape))


def kernel(x, edge_index, edge_type, W_des, b_des, W_num, b_num, W_cat, b_cat,
           W_tot, b_tot, a_emb, W1, b1, W2, b2, Wr, br, Wp1, bp1, Wp2, bp2):
    del edge_type
    f32 = jnp.float32
    n = x.shape[0]
    k = x.shape[1]

    num_idx = jnp.array([4, 6, 7, 8, 10, 11, 12, 13, 14, 15], dtype=jnp.int32)
    cat_idx = jnp.array([1, 2, 3, 5, 9, 16, 17, 18, 19, 20], dtype=jnp.int32)
    wf = jnp.zeros((k, HID), f32)
    wf = wf.at[k - 768:, 0:96].set(W_des)
    wf = wf.at[num_idx, 96:112].set(W_num)
    wf = wf.at[cat_idx, 112:128].set(W_cat)
    ba = jnp.concatenate([b_des, b_num, b_cat]).reshape(1, HID)

    bm = 1000
    grid = (n // bm,)
    row_spec = pl.BlockSpec((bm, HID), lambda i: (i, 0))
    g1, r = pl.pallas_call(
        _stage_a_body,
        grid=grid,
        in_specs=[pl.BlockSpec((bm, k), lambda i: (i, 0)),
                  _full((k, HID)), _full((1, HID)),
                  _full((HID, HID)), _full((1, HID)),
                  _full((HID, HID)), _full((1, HID)),
                  _full((HID, HID)), _full((1, HID)),
                  _full((1, 1))],
        out_specs=[row_spec, row_spec],
        out_shape=[jax.ShapeDtypeStruct((n, HID), f32),
                   jax.ShapeDtypeStruct((n, HID), f32)],
    )(x, wf, ba, W_tot, b_tot.reshape(1, HID), W1, b1.reshape(1, HID),
      Wr, br.reshape(1, HID), a_emb.reshape(1, 1))

    src = edge_index[0]
    dst = edge_index[1]
    n_edges = src.shape[0]
    epad = ((n_edges + NC * NS * EDGE_BLK - 1) // (NC * NS * EDGE_BLK)
            * (NC * NS * EDGE_BLK))
    npad = epad - n_edges
    extra = IDX_ROWS * CHUNK
    pad_i = jnp.arange(npad + extra, dtype=jnp.int32)
    src_p = jnp.concatenate([src, pad_i % N_NODES]).reshape(-1, CHUNK)
    dst_p = jnp.concatenate(
        [dst, N_NODES + pad_i % (ACC_ROWS - N_NODES)]).reshape(-1, CHUNK)

    z128 = jnp.zeros((CHUNK, HID), f32)

    s1, deg1 = _make_seg_kernel(epad, True)(g1, src_p, dst_p, z128)
    s1 = s1.reshape(NC, ACC_ROWS, HID)
    deg1 = deg1.reshape(NC, ACC_ROWS, 1)

    bm2 = 1024
    grid2 = (ACC_ROWS // bm2,)
    g2 = pl.pallas_call(
        _stage_b_body,
        grid=grid2,
        in_specs=[pl.BlockSpec((NC, bm2, HID), lambda i: (0, i, 0)),
                  pl.BlockSpec((NC, bm2, 1), lambda i: (0, i, 0)),
                  _full((HID, HID)), _full((1, HID))],
        out_specs=pl.BlockSpec((bm2, HID), lambda i: (i, 0)),
        out_shape=jax.ShapeDtypeStruct((ACC_ROWS, HID), f32),
    )(s1, deg1, W2, b2.reshape(1, HID))

    s2 = _make_seg_kernel(epad, False)(g2, src_p, dst_p, z128)
    s2 = s2.reshape(NC, ACC_ROWS, HID)

    wp2 = jnp.zeros((HID, HID), f32).at[:, :2].set(Wp2)
    bp2p = jnp.zeros((1, HID), f32).at[0, :2].set(bp2)
    out = pl.pallas_call(
        _stage_c_body,
        grid=grid,
        in_specs=[pl.BlockSpec((NC, bm, HID), lambda i: (0, i, 0)),
                  pl.BlockSpec((NC, bm, 1), lambda i: (0, i, 0)),
                  row_spec,
                  _full((HID, HID)), _full((1, HID)),
                  _full((HID, HID)), _full((1, HID))],
        out_specs=row_spec,
        out_shape=jax.ShapeDtypeStruct((n, HID), f32),
    )(s2, deg1, r, Wp1, bp1.reshape(1, HID), wp2, bp2p)

    return out[:, :2]


# speedup vs baseline: 12.8135x; 1.2045x over previous
import functools

import numpy as np
import jax
import jax.numpy as jnp
from jax import lax
from jax.experimental import pallas as pl
from jax.experimental.pallas import tpu as pltpu
from jax.experimental.pallas import tpu_sc as plsc

N_NODES = 10000
HID = 128
NC = 2
NS = 16
LANES = 128
CHUNK = 64
IDX_ROWS = 8
EDGE_BLK = IDX_ROWS * CHUNK
ACC_ROWS = 10240
ROWS_PER_TILE = ACC_ROWS // NS


def _prelu(x, a):
    return jnp.where(x >= 0, x, a * x)


DEG_PANEL = ACC_ROWS // 8


def _seg_body(nblk, with_deg, table, src_i, dst_i, z128, *refs):
    if with_deg:
        (out, deg_out, acc, dacc, srcv, dstv, rowsa, rowsb,
         ones1, dstage, sema, semb, semc, semd) = refs
    else:
        out, acc, srcv, dstv, rowsa, rowsb, sema, semb, semc, semd = refs
    c = lax.axis_index("c")
    s = lax.axis_index("s")
    row0 = s * ROWS_PER_TILE
    nchunk = ROWS_PER_TILE // CHUNK
    pltpu.sync_copy(z128, rowsa)
    zdescs = [pltpu.async_copy(rowsa, acc.at[pl.ds(row0 + t * CHUNK, CHUNK)],
                               semc)
              for t in range(nchunk)]
    if with_deg:
        zero16 = jnp.zeros((16,), jnp.float32)
        for i in range(ROWS_PER_TILE // 16):
            dstage[pl.ds(i * 16, 16)] = zero16
        pltpu.sync_copy(dstage, dacc.at[pl.ds(row0, ROWS_PER_TILE)])
        one16 = jnp.ones((16,), jnp.float32)
        for i in range(CHUNK // 16):
            ones1[pl.ds(i * 16, 16)] = one16
    for d in zdescs:
        d.wait()
    plsc.subcore_barrier()

    idx_row_base = (c * NS + s) * (nblk * IDX_ROWS)
    bufs = (rowsa, rowsb)
    sems = (sema, semb)
    npair = nblk // 2

    def idx_load(blk, p):
        r = idx_row_base + blk * IDX_ROWS
        return (pltpu.async_copy(src_i.at[pl.ds(r, IDX_ROWS)], srcv.at[p],
                                 semc),
                pltpu.async_copy(dst_i.at[pl.ds(r, IDX_ROWS)], dstv.at[p],
                                 semd))

    idx_load(0, 0)

    def process(p):
        descs = [None] * IDX_ROWS
        descs[0] = pltpu.async_copy(table.at[srcv.at[p, 0]], bufs[0], sems[0])
        for j in range(IDX_ROWS):
            if j + 1 < IDX_ROWS:
                descs[j + 1] = pltpu.async_copy(
                    table.at[srcv.at[p, j + 1]], bufs[(j + 1) % 2],
                    sems[(j + 1) % 2])
            descs[j].wait()
            pltpu.sync_copy(bufs[j % 2], acc.at[dstv.at[p, j]], add=True)
            if with_deg:
                pltpu.sync_copy(ones1, dacc.at[dstv.at[p, j]], add=True)

    def idx_drain(p):
        pltpu.make_async_copy(src_i.at[pl.ds(0, IDX_ROWS)], srcv.at[p],
                              semc).wait()
        pltpu.make_async_copy(dst_i.at[pl.ds(0, IDX_ROWS)], dstv.at[p],
                              semd).wait()

    @pl.loop(0, npair)
    def _outer(bb):
        b0 = bb * 2
        idx_load(b0 + 1, 1)
        idx_drain(0)
        process(0)
        idx_load(b0 + 2, 0)
        idx_drain(1)
        process(1)

    idx_drain(0)
    plsc.subcore_barrier()
    base = c * ACC_ROWS + row0
    loads = [None] * nchunk
    outs = [None] * nchunk
    loads[0] = pltpu.async_copy(acc.at[pl.ds(row0, CHUNK)], rowsa, sema)
    for t in range(nchunk):
        bt = bufs[t % 2]
        if t + 1 < nchunk:
            if t >= 1:
                outs[t - 1].wait()
            loads[t + 1] = pltpu.async_copy(
                acc.at[pl.ds(row0 + (t + 1) * CHUNK, CHUNK)],
                bufs[(t + 1) % 2], sems[(t + 1) % 2])
        loads[t].wait()
        outs[t] = pltpu.async_copy(bt, out.at[pl.ds(base + t * CHUNK, CHUNK)],
                                   semc if t % 2 == 0 else semd)
    outs[nchunk - 2].wait()
    outs[nchunk - 1].wait()
    if with_deg:
        pltpu.sync_copy(dacc.at[pl.ds(row0, ROWS_PER_TILE)], dstage)
        pltpu.sync_copy(dstage, deg_out.at[pl.ds(base, ROWS_PER_TILE)])


def _make_seg_kernel(n_edges_pad, with_deg):
    nblk = n_edges_pad // (NC * NS * EDGE_BLK)
    mesh = plsc.VectorSubcoreMesh(core_axis_name="c", subcore_axis_name="s")
    outs = [jax.ShapeDtypeStruct((NC * ACC_ROWS, HID), jnp.float32)]
    scratch = [
        pltpu.VMEM_SHARED((ACC_ROWS, HID), jnp.float32),
    ]
    if with_deg:
        outs.append(jax.ShapeDtypeStruct((NC * ACC_ROWS,), jnp.float32))
        scratch.append(pltpu.VMEM_SHARED((ACC_ROWS,), jnp.float32))
    scratch += [
        pltpu.VMEM((2, IDX_ROWS, CHUNK), jnp.int32),
        pltpu.VMEM((2, IDX_ROWS, CHUNK), jnp.int32),
        pltpu.VMEM((CHUNK, HID), jnp.float32),
        pltpu.VMEM((CHUNK, HID), jnp.float32),
    ]
    if with_deg:
        scratch += [
            pltpu.VMEM((CHUNK,), jnp.float32),
            pltpu.VMEM((ROWS_PER_TILE,), jnp.float32),
        ]
    scratch += [pltpu.SemaphoreType.DMA] * 4
    return pl.kernel(
        functools.partial(_seg_body, nblk, with_deg),
        out_type=tuple(outs) if with_deg else outs[0],
        mesh=mesh,
        scratch_types=scratch,
    )


def _stage_a_body(x, wf, ba, wt, bt, w1, b1, wr, br, a, g1, r):
    av = a[0, 0]
    e = _prelu(jnp.dot(x[...], wf[...], preferred_element_type=jnp.float32)
               + ba[...], av)
    h0 = _prelu(jnp.dot(e, wt[...], preferred_element_type=jnp.float32)
                + bt[...], av)
    g1[...] = jnp.dot(h0, w1[...], preferred_element_type=jnp.float32) + b1[...]
    r[...] = jnp.dot(h0, wr[...], preferred_element_type=jnp.float32) + br[...]


def _stage_b_body(sp, dp, w2, b2, g2):
    ssum = sp[0] + sp[1]
    deg = jnp.maximum(dp[0] + dp[1], 1.0)
    h = jnp.maximum(ssum / deg, 0.0)
    g2[...] = jnp.dot(h, w2[...], preferred_element_type=jnp.float32) + b2[...]


def _stage_c_body(sp, dp, r, wp1, bp1, wp2, bp2, out):
    ssum = sp[0] + sp[1]
    deg = jnp.maximum(dp[0] + dp[1], 1.0)
    h2 = jnp.maximum(ssum / deg, 0.0)
    t = h2 + r[...]
    p = jnp.maximum(jnp.dot(t, wp1[...], preferred_element_type=jnp.float32)
                    + bp1[...], 0.0)
    out[...] = jnp.dot(p, wp2[...], preferred_element_type=jnp.float32) + bp2[...]


def _full(shape):
    return pl.BlockSpec(shape, lambda i: tuple(0 for _ in shape))


def kernel(x, edge_index, edge_type, W_des, b_des, W_num, b_num, W_cat, b_cat,
           W_tot, b_tot, a_emb, W1, b1, W2, b2, Wr, br, Wp1, bp1, Wp2, bp2):
    del edge_type
    f32 = jnp.float32
    n = x.shape[0]
    k = x.shape[1]

    num_idx = np.array([4, 6, 7, 8, 10, 11, 12, 13, 14, 15])
    cat_idx = np.array([1, 2, 3, 5, 9, 16, 17, 18, 19, 20])
    ktop = k - 768
    m_num = np.zeros((ktop, 10), np.float32)
    m_num[num_idx, np.arange(10)] = 1.0
    m_cat = np.zeros((ktop, 10), np.float32)
    m_cat[cat_idx, np.arange(10)] = 1.0
    top = jnp.concatenate(
        [jnp.zeros((ktop, 96), f32), jnp.asarray(m_num) @ W_num,
         jnp.asarray(m_cat) @ W_cat], axis=1)
    bot = jnp.concatenate([W_des, jnp.zeros((768, 32), f32)], axis=1)
    wf = jnp.concatenate([top, bot], axis=0)
    ba = jnp.concatenate([b_des, b_num, b_cat]).reshape(1, HID)

    bm = 1000
    grid = (n // bm,)
    row_spec = pl.BlockSpec((bm, HID), lambda i: (i, 0))
    g1, r = pl.pallas_call(
        _stage_a_body,
        grid=grid,
        in_specs=[pl.BlockSpec((bm, k), lambda i: (i, 0)),
                  _full((k, HID)), _full((1, HID)),
                  _full((HID, HID)), _full((1, HID)),
                  _full((HID, HID)), _full((1, HID)),
                  _full((HID, HID)), _full((1, HID)),
                  _full((1, 1))],
        out_specs=[row_spec, row_spec],
        out_shape=[jax.ShapeDtypeStruct((n, HID), f32),
                   jax.ShapeDtypeStruct((n, HID), f32)],
    )(x, wf, ba, W_tot, b_tot.reshape(1, HID), W1, b1.reshape(1, HID),
      Wr, br.reshape(1, HID), a_emb.reshape(1, 1))

    src = edge_index[0]
    dst = edge_index[1]
    n_edges = src.shape[0]
    epad = ((n_edges + NC * NS * EDGE_BLK - 1) // (NC * NS * EDGE_BLK)
            * (NC * NS * EDGE_BLK))
    npad = epad - n_edges
    extra = IDX_ROWS * CHUNK
    pad_i = np.arange(npad + extra, dtype=np.int32)
    pad_src = jnp.asarray(pad_i % N_NODES)
    pad_dst = jnp.asarray(N_NODES + pad_i % (ACC_ROWS - N_NODES))
    src_p = jnp.concatenate([src, pad_src]).reshape(-1, CHUNK)
    dst_p = jnp.concatenate([dst, pad_dst]).reshape(-1, CHUNK)

    z128 = jnp.zeros((CHUNK, HID), f32)

    s1, deg1 = _make_seg_kernel(epad, True)(g1, src_p, dst_p, z128)
    s1 = s1.reshape(NC, ACC_ROWS, HID)
    deg1 = deg1.reshape(NC, ACC_ROWS, 1)

    bm2 = 1024
    grid2 = (ACC_ROWS // bm2,)
    g2 = pl.pallas_call(
        _stage_b_body,
        grid=grid2,
        in_specs=[pl.BlockSpec((NC, bm2, HID), lambda i: (0, i, 0)),
                  pl.BlockSpec((NC, bm2, 1), lambda i: (0, i, 0)),
                  _full((HID, HID)), _full((1, HID))],
        out_specs=pl.BlockSpec((bm2, HID), lambda i: (i, 0)),
        out_shape=jax.ShapeDtypeStruct((ACC_ROWS, HID), f32),
    )(s1, deg1, W2, b2.reshape(1, HID))

    s2 = _make_seg_kernel(epad, False)(g2, src_p, dst_p, z128)
    s2 = s2.reshape(NC, ACC_ROWS, HID)

    wp2 = jnp.zeros((HID, HID), f32).at[:, :2].set(Wp2)
    bp2p = jnp.zeros((1, HID), f32).at[0, :2].set(bp2)
    out = pl.pallas_call(
        _stage_c_body,
        grid=grid,
        in_specs=[pl.BlockSpec((NC, bm, HID), lambda i: (0, i, 0)),
                  pl.BlockSpec((NC, bm, 1), lambda i: (0, i, 0)),
                  row_spec,
                  _full((HID, HID)), _full((1, HID)),
                  _full((HID, HID)), _full((1, HID))],
        out_specs=row_spec,
        out_shape=jax.ShapeDtypeStruct((n, HID), f32),
    )(s2, deg1, r, Wp1, bp1.reshape(1, HID), wp2, bp2p)

    return out[:, :2]


# speedup vs baseline: 14.1537x; 1.1046x over previous
import functools

import numpy as np
import jax
import jax.numpy as jnp
from jax import lax
from jax.experimental import pallas as pl
from jax.experimental.pallas import tpu as pltpu
from jax.experimental.pallas import tpu_sc as plsc

N_NODES = 10000
HID = 128
NC = 2
NS = 16
LANES = 128
CHUNK = 64
IDX_ROWS = 8
EDGE_BLK = IDX_ROWS * CHUNK
ACC_ROWS = 10240
ROWS_PER_TILE = ACC_ROWS // NS


def _prelu(x, a):
    return jnp.where(x >= 0, x, a * x)


DEG_PANEL = ACC_ROWS // 8


def _seg_body(nblk, with_deg, table, src_i, dst_i, z128, *refs):
    if with_deg:
        (out, deg_out, acc, dacc, srcv, dstv, rowsa, rowsb, rowsc,
         ones1, dstage, sema, semb, semc, semd, seme) = refs
    else:
        (out, acc, srcv, dstv, rowsa, rowsb, rowsc,
         sema, semb, semc, semd, seme) = refs
    c = lax.axis_index("c")
    s = lax.axis_index("s")
    row0 = s * ROWS_PER_TILE
    nchunk = ROWS_PER_TILE // CHUNK
    pltpu.sync_copy(z128, rowsa)
    zdescs = [pltpu.async_copy(rowsa, acc.at[pl.ds(row0 + t * CHUNK, CHUNK)],
                               semc)
              for t in range(nchunk)]
    if with_deg:
        zero16 = jnp.zeros((16,), jnp.float32)
        for i in range(ROWS_PER_TILE // 16):
            dstage[pl.ds(i * 16, 16)] = zero16
        pltpu.sync_copy(dstage, dacc.at[pl.ds(row0, ROWS_PER_TILE)])
        one16 = jnp.ones((16,), jnp.float32)
        for i in range(CHUNK // 16):
            ones1[pl.ds(i * 16, 16)] = one16
    for d in zdescs:
        d.wait()
    plsc.subcore_barrier()

    idx_row_base = (c * NS + s) * (nblk * IDX_ROWS)
    bufs = (rowsa, rowsb, rowsc)
    sems = (sema, semb, seme)
    npair = nblk // 2

    def idx_load(blk, p):
        r = idx_row_base + blk * IDX_ROWS
        return (pltpu.async_copy(src_i.at[pl.ds(r, IDX_ROWS)], srcv.at[p],
                                 semc),
                pltpu.async_copy(dst_i.at[pl.ds(r, IDX_ROWS)], dstv.at[p],
                                 semd))

    idx_load(0, 0)

    def process(p):
        descs = [None] * IDX_ROWS
        descs[0] = pltpu.async_copy(table.at[srcv.at[p, 0]], bufs[0], sems[0])
        descs[1] = pltpu.async_copy(table.at[srcv.at[p, 1]], bufs[1], sems[1])
        for j in range(IDX_ROWS):
            if j + 2 < IDX_ROWS:
                descs[j + 2] = pltpu.async_copy(
                    table.at[srcv.at[p, j + 2]], bufs[(j + 2) % 3],
                    sems[(j + 2) % 3])
            descs[j].wait()
            pltpu.sync_copy(bufs[j % 3], acc.at[dstv.at[p, j]], add=True)
            if with_deg:
                pltpu.sync_copy(ones1, dacc.at[dstv.at[p, j]], add=True)

    def idx_drain(p):
        pltpu.make_async_copy(src_i.at[pl.ds(0, IDX_ROWS)], srcv.at[p],
                              semc).wait()
        pltpu.make_async_copy(dst_i.at[pl.ds(0, IDX_ROWS)], dstv.at[p],
                              semd).wait()

    @pl.loop(0, npair)
    def _outer(bb):
        b0 = bb * 2
        idx_load(b0 + 1, 1)
        idx_drain(0)
        process(0)
        idx_load(b0 + 2, 0)
        idx_drain(1)
        process(1)

    idx_drain(0)
    plsc.subcore_barrier()
    base = c * ACC_ROWS + row0
    loads = [None] * nchunk
    outs = [None] * nchunk
    loads[0] = pltpu.async_copy(acc.at[pl.ds(row0, CHUNK)], rowsa, sema)
    for t in range(nchunk):
        bt = bufs[t % 2]
        if t + 1 < nchunk:
            if t >= 1:
                outs[t - 1].wait()
            loads[t + 1] = pltpu.async_copy(
                acc.at[pl.ds(row0 + (t + 1) * CHUNK, CHUNK)],
                bufs[(t + 1) % 2], sems[(t + 1) % 2])
        loads[t].wait()
        outs[t] = pltpu.async_copy(bt, out.at[pl.ds(base + t * CHUNK, CHUNK)],
                                   semc if t % 2 == 0 else semd)
    outs[nchunk - 2].wait()
    outs[nchunk - 1].wait()
    if with_deg:
        pltpu.sync_copy(dacc.at[pl.ds(row0, ROWS_PER_TILE)], dstage)
        pltpu.sync_copy(dstage, deg_out.at[pl.ds(base, ROWS_PER_TILE)])


def _make_seg_kernel(n_edges_pad, with_deg):
    nblk = n_edges_pad // (NC * NS * EDGE_BLK)
    mesh = plsc.VectorSubcoreMesh(core_axis_name="c", subcore_axis_name="s")
    outs = [jax.ShapeDtypeStruct((NC * ACC_ROWS, HID), jnp.float32)]
    scratch = [
        pltpu.VMEM_SHARED((ACC_ROWS, HID), jnp.float32),
    ]
    if with_deg:
        outs.append(jax.ShapeDtypeStruct((NC * ACC_ROWS,), jnp.float32))
        scratch.append(pltpu.VMEM_SHARED((ACC_ROWS,), jnp.float32))
    scratch += [
        pltpu.VMEM((2, IDX_ROWS, CHUNK), jnp.int32),
        pltpu.VMEM((2, IDX_ROWS, CHUNK), jnp.int32),
        pltpu.VMEM((CHUNK, HID), jnp.float32),
        pltpu.VMEM((CHUNK, HID), jnp.float32),
        pltpu.VMEM((CHUNK, HID), jnp.float32),
    ]
    if with_deg:
        scratch += [
            pltpu.VMEM((CHUNK,), jnp.float32),
            pltpu.VMEM((ROWS_PER_TILE,), jnp.float32),
        ]
    scratch += [pltpu.SemaphoreType.DMA] * 5
    return pl.kernel(
        functools.partial(_seg_body, nblk, with_deg),
        out_type=tuple(outs) if with_deg else outs[0],
        mesh=mesh,
        scratch_types=scratch,
    )


def _stage_a_body(x, wf, ba, wt, bt, w1, b1, wr, br, a, g1, r):
    av = a[0, 0]
    e = _prelu(jnp.dot(x[...], wf[...], preferred_element_type=jnp.float32)
               + ba[...], av)
    h0 = _prelu(jnp.dot(e, wt[...], preferred_element_type=jnp.float32)
                + bt[...], av)
    g1[...] = jnp.dot(h0, w1[...], preferred_element_type=jnp.float32) + b1[...]
    r[...] = jnp.dot(h0, wr[...], preferred_element_type=jnp.float32) + br[...]


def _stage_b_body(sp, dp, w2, b2, g2):
    ssum = sp[0] + sp[1]
    deg = jnp.maximum(dp[0] + dp[1], 1.0)
    h = jnp.maximum(ssum / deg, 0.0)
    g2[...] = jnp.dot(h, w2[...], preferred_element_type=jnp.float32) + b2[...]


def _stage_c_body(sp, dp, r, wp1, bp1, wp2, bp2, out):
    ssum = sp[0] + sp[1]
    deg = jnp.maximum(dp[0] + dp[1], 1.0)
    h2 = jnp.maximum(ssum / deg, 0.0)
    t = h2 + r[...]
    p = jnp.maximum(jnp.dot(t, wp1[...], preferred_element_type=jnp.float32)
                    + bp1[...], 0.0)
    out[...] = jnp.dot(p, wp2[...], preferred_element_type=jnp.float32) + bp2[...]


def _full(shape):
    return pl.BlockSpec(shape, lambda i: tuple(0 for _ in shape))


def kernel(x, edge_index, edge_type, W_des, b_des, W_num, b_num, W_cat, b_cat,
           W_tot, b_tot, a_emb, W1, b1, W2, b2, Wr, br, Wp1, bp1, Wp2, bp2):
    del edge_type
    f32 = jnp.float32
    n = x.shape[0]
    k = x.shape[1]

    num_idx = np.array([4, 6, 7, 8, 10, 11, 12, 13, 14, 15])
    cat_idx = np.array([1, 2, 3, 5, 9, 16, 17, 18, 19, 20])
    ktop = k - 768
    m_num = np.zeros((ktop, 10), np.float32)
    m_num[num_idx, np.arange(10)] = 1.0
    m_cat = np.zeros((ktop, 10), np.float32)
    m_cat[cat_idx, np.arange(10)] = 1.0
    top = jnp.concatenate(
        [jnp.zeros((ktop, 96), f32), jnp.asarray(m_num) @ W_num,
         jnp.asarray(m_cat) @ W_cat], axis=1)
    bot = jnp.concatenate([W_des, jnp.zeros((768, 32), f32)], axis=1)
    wf = jnp.concatenate([top, bot], axis=0)
    ba = jnp.concatenate([b_des, b_num, b_cat]).reshape(1, HID)

    bm = 1000
    grid = (n // bm,)
    row_spec = pl.BlockSpec((bm, HID), lambda i: (i, 0))
    g1, r = pl.pallas_call(
        _stage_a_body,
        grid=grid,
        in_specs=[pl.BlockSpec((bm, k), lambda i: (i, 0)),
                  _full((k, HID)), _full((1, HID)),
                  _full((HID, HID)), _full((1, HID)),
                  _full((HID, HID)), _full((1, HID)),
                  _full((HID, HID)), _full((1, HID)),
                  _full((1, 1))],
        out_specs=[row_spec, row_spec],
        out_shape=[jax.ShapeDtypeStruct((n, HID), f32),
                   jax.ShapeDtypeStruct((n, HID), f32)],
    )(x, wf, ba, W_tot, b_tot.reshape(1, HID), W1, b1.reshape(1, HID),
      Wr, br.reshape(1, HID), a_emb.reshape(1, 1))

    src = edge_index[0]
    dst = edge_index[1]
    n_edges = src.shape[0]
    epad = ((n_edges + NC * NS * EDGE_BLK - 1) // (NC * NS * EDGE_BLK)
            * (NC * NS * EDGE_BLK))
    npad = epad - n_edges
    extra = IDX_ROWS * CHUNK
    pad_i = np.arange(npad + extra, dtype=np.int32)
    pad_src = jnp.asarray(pad_i % N_NODES)
    pad_dst = jnp.asarray(N_NODES + pad_i % (ACC_ROWS - N_NODES))
    src_p = jnp.concatenate([src, pad_src]).reshape(-1, CHUNK)
    dst_p = jnp.concatenate([dst, pad_dst]).reshape(-1, CHUNK)

    z128 = jnp.zeros((CHUNK, HID), f32)

    s1, deg1 = _make_seg_kernel(epad, True)(g1, src_p, dst_p, z128)
    s1 = s1.reshape(NC, ACC_ROWS, HID)
    deg1 = deg1.reshape(NC, ACC_ROWS, 1)

    bm2 = 1024
    grid2 = (ACC_ROWS // bm2,)
    g2 = pl.pallas_call(
        _stage_b_body,
        grid=grid2,
        in_specs=[pl.BlockSpec((NC, bm2, HID), lambda i: (0, i, 0)),
                  pl.BlockSpec((NC, bm2, 1), lambda i: (0, i, 0)),
                  _full((HID, HID)), _full((1, HID))],
        out_specs=pl.BlockSpec((bm2, HID), lambda i: (i, 0)),
        out_shape=jax.ShapeDtypeStruct((ACC_ROWS, HID), f32),
    )(s1, deg1, W2, b2.reshape(1, HID))

    s2 = _make_seg_kernel(epad, False)(g2, src_p, dst_p, z128)
    s2 = s2.reshape(NC, ACC_ROWS, HID)

    wp2 = jnp.zeros((HID, HID), f32).at[:, :2].set(Wp2)
    bp2p = jnp.zeros((1, HID), f32).at[0, :2].set(bp2)
    out = pl.pallas_call(
        _stage_c_body,
        grid=grid,
        in_specs=[pl.BlockSpec((NC, bm, HID), lambda i: (0, i, 0)),
                  pl.BlockSpec((NC, bm, 1), lambda i: (0, i, 0)),
                  row_spec,
                  _full((HID, HID)), _full((1, HID)),
                  _full((HID, HID)), _full((1, HID))],
        out_specs=row_spec,
        out_shape=jax.ShapeDtypeStruct((n, HID), f32),
    )(s2, deg1, r, Wp1, bp1.reshape(1, HID), wp2, bp2p)

    return out[:, :2]


# speedup vs baseline: 14.3258x; 1.0122x over previous
import functools

import numpy as np
import jax
import jax.numpy as jnp
from jax import lax
from jax.experimental import pallas as pl
from jax.experimental.pallas import tpu as pltpu
from jax.experimental.pallas import tpu_sc as plsc

N_NODES = 10000
HID = 128
NC = 2
NS = 16
LANES = 128
CHUNK = 64
IDX_ROWS = 8
EDGE_BLK = IDX_ROWS * CHUNK
ACC_ROWS = 10240
ROWS_PER_TILE = ACC_ROWS // NS


def _prelu(x, a):
    return jnp.where(x >= 0, x, a * x)


DEG_PANEL = ACC_ROWS // 8


def _seg_body(nblk, with_deg, table, src_i, dst_i, z128, *refs):
    if with_deg:
        (out, deg_out, acc, dacc, srcv, dstv, rowsa, rowsb, rowsc,
         ones1, dstage, sema, semb, semc, semd, seme,
         semf, semg, semh) = refs
    else:
        (out, acc, srcv, dstv, rowsa, rowsb, rowsc,
         sema, semb, semc, semd, seme, semf, semg, semh) = refs
    c = lax.axis_index("c")
    s = lax.axis_index("s")
    row0 = s * ROWS_PER_TILE
    nchunk = ROWS_PER_TILE // CHUNK
    pltpu.sync_copy(z128, rowsa)
    zdescs = [pltpu.async_copy(rowsa, acc.at[pl.ds(row0 + t * CHUNK, CHUNK)],
                               semc)
              for t in range(nchunk)]
    if with_deg:
        zero16 = jnp.zeros((16,), jnp.float32)
        for i in range(ROWS_PER_TILE // 16):
            dstage[pl.ds(i * 16, 16)] = zero16
        pltpu.sync_copy(dstage, dacc.at[pl.ds(row0, ROWS_PER_TILE)])
        one16 = jnp.ones((16,), jnp.float32)
        for i in range(CHUNK // 16):
            ones1[pl.ds(i * 16, 16)] = one16
    for d in zdescs:
        d.wait()
    plsc.subcore_barrier()

    idx_row_base = (c * NS + s) * (nblk * IDX_ROWS)
    bufs = (rowsa, rowsb, rowsc)
    sems = (sema, semb, seme)
    npair = nblk // 2

    def idx_load(blk, p):
        r = idx_row_base + blk * IDX_ROWS
        return (pltpu.async_copy(src_i.at[pl.ds(r, IDX_ROWS)], srcv.at[p],
                                 semc),
                pltpu.async_copy(dst_i.at[pl.ds(r, IDX_ROWS)], dstv.at[p],
                                 semd))

    idx_load(0, 0)

    ssems = (semf, semg, semh)

    def process(p):
        descs = [None] * IDX_ROWS
        sdescs = [None] * IDX_ROWS
        descs[0] = pltpu.async_copy(table.at[srcv.at[p, 0]], bufs[0], sems[0])
        descs[1] = pltpu.async_copy(table.at[srcv.at[p, 1]], bufs[1], sems[1])
        for j in range(IDX_ROWS):
            if j + 2 < IDX_ROWS:
                if j >= 1:
                    sdescs[j - 1].wait()
                descs[j + 2] = pltpu.async_copy(
                    table.at[srcv.at[p, j + 2]], bufs[(j + 2) % 3],
                    sems[(j + 2) % 3])
            descs[j].wait()
            sdescs[j] = pltpu.async_copy(bufs[j % 3], acc.at[dstv.at[p, j]],
                                         ssems[j % 3], add=True)
            if with_deg:
                pltpu.sync_copy(ones1, dacc.at[dstv.at[p, j]], add=True)
        for j in range(max(IDX_ROWS - 3, 1), IDX_ROWS):
            sdescs[j].wait()

    def idx_drain(p):
        pltpu.make_async_copy(src_i.at[pl.ds(0, IDX_ROWS)], srcv.at[p],
                              semc).wait()
        pltpu.make_async_copy(dst_i.at[pl.ds(0, IDX_ROWS)], dstv.at[p],
                              semd).wait()

    @pl.loop(0, npair)
    def _outer(bb):
        b0 = bb * 2
        idx_load(b0 + 1, 1)
        idx_drain(0)
        process(0)
        idx_load(b0 + 2, 0)
        idx_drain(1)
        process(1)

    idx_drain(0)
    plsc.subcore_barrier()
    base = c * ACC_ROWS + row0
    loads = [None] * nchunk
    outs = [None] * nchunk
    loads[0] = pltpu.async_copy(acc.at[pl.ds(row0, CHUNK)], rowsa, sema)
    for t in range(nchunk):
        bt = bufs[t % 2]
        if t + 1 < nchunk:
            if t >= 1:
                outs[t - 1].wait()
            loads[t + 1] = pltpu.async_copy(
                acc.at[pl.ds(row0 + (t + 1) * CHUNK, CHUNK)],
                bufs[(t + 1) % 2], sems[(t + 1) % 2])
        loads[t].wait()
        outs[t] = pltpu.async_copy(bt, out.at[pl.ds(base + t * CHUNK, CHUNK)],
                                   semc if t % 2 == 0 else semd)
    outs[nchunk - 2].wait()
    outs[nchunk - 1].wait()
    if with_deg:
        pltpu.sync_copy(dacc.at[pl.ds(row0, ROWS_PER_TILE)], dstage)
        pltpu.sync_copy(dstage, deg_out.at[pl.ds(base, ROWS_PER_TILE)])


def _make_seg_kernel(n_edges_pad, with_deg):
    nblk = n_edges_pad // (NC * NS * EDGE_BLK)
    mesh = plsc.VectorSubcoreMesh(core_axis_name="c", subcore_axis_name="s")
    outs = [jax.ShapeDtypeStruct((NC * ACC_ROWS, HID), jnp.float32)]
    scratch = [
        pltpu.VMEM_SHARED((ACC_ROWS, HID), jnp.float32),
    ]
    if with_deg:
        outs.append(jax.ShapeDtypeStruct((NC * ACC_ROWS,), jnp.float32))
        scratch.append(pltpu.VMEM_SHARED((ACC_ROWS,), jnp.float32))
    scratch += [
        pltpu.VMEM((2, IDX_ROWS, CHUNK), jnp.int32),
        pltpu.VMEM((2, IDX_ROWS, CHUNK), jnp.int32),
        pltpu.VMEM((CHUNK, HID), jnp.float32),
        pltpu.VMEM((CHUNK, HID), jnp.float32),
        pltpu.VMEM((CHUNK, HID), jnp.float32),
    ]
    if with_deg:
        scratch += [
            pltpu.VMEM((CHUNK,), jnp.float32),
            pltpu.VMEM((ROWS_PER_TILE,), jnp.float32),
        ]
    scratch += [pltpu.SemaphoreType.DMA] * 8
    return pl.kernel(
        functools.partial(_seg_body, nblk, with_deg),
        out_type=tuple(outs) if with_deg else outs[0],
        mesh=mesh,
        scratch_types=scratch,
    )


def _stage_a_body(x, wf, ba, wt, bt, w1, b1, wr, br, a, g1, r):
    av = a[0, 0]
    e = _prelu(jnp.dot(x[...], wf[...], preferred_element_type=jnp.float32)
               + ba[...], av)
    h0 = _prelu(jnp.dot(e, wt[...], preferred_element_type=jnp.float32)
                + bt[...], av)
    g1[...] = jnp.dot(h0, w1[...], preferred_element_type=jnp.float32) + b1[...]
    r[...] = jnp.dot(h0, wr[...], preferred_element_type=jnp.float32) + br[...]


def _stage_b_body(sp, dp, w2, b2, g2):
    ssum = sp[0] + sp[1]
    deg = jnp.maximum(dp[0] + dp[1], 1.0)
    h = jnp.maximum(ssum / deg, 0.0)
    g2[...] = jnp.dot(h, w2[...], preferred_element_type=jnp.float32) + b2[...]


def _stage_c_body(sp, dp, r, wp1, bp1, wp2, bp2, out):
    ssum = sp[0] + sp[1]
    deg = jnp.maximum(dp[0] + dp[1], 1.0)
    h2 = jnp.maximum(ssum / deg, 0.0)
    t = h2 + r[...]
    p = jnp.maximum(jnp.dot(t, wp1[...], preferred_element_type=jnp.float32)
                    + bp1[...], 0.0)
    out[...] = jnp.dot(p, wp2[...], preferred_element_type=jnp.float32) + bp2[...]


def _full(shape):
    return pl.BlockSpec(shape, lambda i: tuple(0 for _ in shape))


def kernel(x, edge_index, edge_type, W_des, b_des, W_num, b_num, W_cat, b_cat,
           W_tot, b_tot, a_emb, W1, b1, W2, b2, Wr, br, Wp1, bp1, Wp2, bp2):
    del edge_type
    f32 = jnp.float32
    n = x.shape[0]
    k = x.shape[1]

    num_idx = np.array([4, 6, 7, 8, 10, 11, 12, 13, 14, 15])
    cat_idx = np.array([1, 2, 3, 5, 9, 16, 17, 18, 19, 20])
    ktop = k - 768
    m_num = np.zeros((ktop, 10), np.float32)
    m_num[num_idx, np.arange(10)] = 1.0
    m_cat = np.zeros((ktop, 10), np.float32)
    m_cat[cat_idx, np.arange(10)] = 1.0
    top = jnp.concatenate(
        [jnp.zeros((ktop, 96), f32), jnp.asarray(m_num) @ W_num,
         jnp.asarray(m_cat) @ W_cat], axis=1)
    bot = jnp.concatenate([W_des, jnp.zeros((768, 32), f32)], axis=1)
    wf = jnp.concatenate([top, bot], axis=0)
    ba = jnp.concatenate([b_des, b_num, b_cat]).reshape(1, HID)

    bm = 1000
    grid = (n // bm,)
    row_spec = pl.BlockSpec((bm, HID), lambda i: (i, 0))
    g1, r = pl.pallas_call(
        _stage_a_body,
        grid=grid,
        in_specs=[pl.BlockSpec((bm, k), lambda i: (i, 0)),
                  _full((k, HID)), _full((1, HID)),
                  _full((HID, HID)), _full((1, HID)),
                  _full((HID, HID)), _full((1, HID)),
                  _full((HID, HID)), _full((1, HID)),
                  _full((1, 1))],
        out_specs=[row_spec, row_spec],
        out_shape=[jax.ShapeDtypeStruct((n, HID), f32),
                   jax.ShapeDtypeStruct((n, HID), f32)],
    )(x, wf, ba, W_tot, b_tot.reshape(1, HID), W1, b1.reshape(1, HID),
      Wr, br.reshape(1, HID), a_emb.reshape(1, 1))

    src = edge_index[0]
    dst = edge_index[1]
    n_edges = src.shape[0]
    epad = ((n_edges + NC * NS * EDGE_BLK - 1) // (NC * NS * EDGE_BLK)
            * (NC * NS * EDGE_BLK))
    npad = epad - n_edges
    extra = IDX_ROWS * CHUNK
    pad_i = np.arange(npad + extra, dtype=np.int32)
    pad_src = jnp.asarray(pad_i % N_NODES)
    pad_dst = jnp.asarray(N_NODES + pad_i % (ACC_ROWS - N_NODES))
    src_p = jnp.concatenate([src, pad_src]).reshape(-1, CHUNK)
    dst_p = jnp.concatenate([dst, pad_dst]).reshape(-1, CHUNK)

    z128 = jnp.zeros((CHUNK, HID), f32)

    s1, deg1 = _make_seg_kernel(epad, True)(g1, src_p, dst_p, z128)
    s1 = s1.reshape(NC, ACC_ROWS, HID)
    deg1 = deg1.reshape(NC, ACC_ROWS, 1)

    bm2 = 1024
    grid2 = (ACC_ROWS // bm2,)
    g2 = pl.pallas_call(
        _stage_b_body,
        grid=grid2,
        in_specs=[pl.BlockSpec((NC, bm2, HID), lambda i: (0, i, 0)),
                  pl.BlockSpec((NC, bm2, 1), lambda i: (0, i, 0)),
                  _full((HID, HID)), _full((1, HID))],
        out_specs=pl.BlockSpec((bm2, HID), lambda i: (i, 0)),
        out_shape=jax.ShapeDtypeStruct((ACC_ROWS, HID), f32),
    )(s1, deg1, W2, b2.reshape(1, HID))

    s2 = _make_seg_kernel(epad, False)(g2, src_p, dst_p, z128)
    s2 = s2.reshape(NC, ACC_ROWS, HID)

    wp2 = jnp.zeros((HID, HID), f32).at[:, :2].set(Wp2)
    bp2p = jnp.zeros((1, HID), f32).at[0, :2].set(bp2)
    out = pl.pallas_call(
        _stage_c_body,
        grid=grid,
        in_specs=[pl.BlockSpec((NC, bm, HID), lambda i: (0, i, 0)),
                  pl.BlockSpec((NC, bm, 1), lambda i: (0, i, 0)),
                  row_spec,
                  _full((HID, HID)), _full((1, HID)),
                  _full((HID, HID)), _full((1, HID))],
        out_specs=row_spec,
        out_shape=jax.ShapeDtypeStruct((n, HID), f32),
    )(s2, deg1, r, Wp1, bp1.reshape(1, HID), wp2, bp2p)

    return out[:, :2]


# speedup vs baseline: 14.7731x; 1.0312x over previous
import functools

import numpy as np
import jax
import jax.numpy as jnp
from jax import lax
from jax.experimental import pallas as pl
from jax.experimental.pallas import tpu as pltpu
from jax.experimental.pallas import tpu_sc as plsc

N_NODES = 10000
HID = 128
NC = 2
NS = 16
LANES = 128
CHUNK = 80
IDX_ROWS = 8
EDGE_BLK = IDX_ROWS * CHUNK
ACC_ROWS = 10240
ROWS_PER_TILE = ACC_ROWS // NS


def _prelu(x, a):
    return jnp.where(x >= 0, x, a * x)


DEG_PANEL = ACC_ROWS // 8


def _seg_body(nblk, with_deg, table, src_i, dst_i, z128, *refs):
    if with_deg:
        (out, deg_out, acc, dacc, srcv, dstv, rowsa, rowsb, rowsc,
         ones1, dstage, sema, semb, semc, semd, seme,
         semf, semg, semh) = refs
    else:
        (out, acc, srcv, dstv, rowsa, rowsb, rowsc,
         sema, semb, semc, semd, seme, semf, semg, semh) = refs
    c = lax.axis_index("c")
    s = lax.axis_index("s")
    row0 = s * ROWS_PER_TILE
    nchunk = ROWS_PER_TILE // CHUNK
    pltpu.sync_copy(z128, rowsa)
    zdescs = [pltpu.async_copy(rowsa, acc.at[pl.ds(row0 + t * CHUNK, CHUNK)],
                               semc)
              for t in range(nchunk)]
    if with_deg:
        zero16 = jnp.zeros((16,), jnp.float32)
        for i in range(ROWS_PER_TILE // 16):
            dstage[pl.ds(i * 16, 16)] = zero16
        pltpu.sync_copy(dstage, dacc.at[pl.ds(row0, ROWS_PER_TILE)])
        one16 = jnp.ones((16,), jnp.float32)
        for i in range(CHUNK // 16):
            ones1[pl.ds(i * 16, 16)] = one16
    for d in zdescs:
        d.wait()
    plsc.subcore_barrier()

    idx_row_base = (c * NS + s) * (nblk * IDX_ROWS)
    bufs = (rowsa, rowsb, rowsc)
    sems = (sema, semb, seme)
    npair = nblk // 2

    def idx_load(blk, p):
        r = idx_row_base + blk * IDX_ROWS
        return (pltpu.async_copy(src_i.at[pl.ds(r, IDX_ROWS)], srcv.at[p],
                                 semc),
                pltpu.async_copy(dst_i.at[pl.ds(r, IDX_ROWS)], dstv.at[p],
                                 semd))

    idx_load(0, 0)

    ssems = (semf, semg, semh)

    def process(p):
        descs = [None] * IDX_ROWS
        sdescs = [None] * IDX_ROWS
        descs[0] = pltpu.async_copy(table.at[srcv.at[p, 0]], bufs[0], sems[0])
        descs[1] = pltpu.async_copy(table.at[srcv.at[p, 1]], bufs[1], sems[1])
        for j in range(IDX_ROWS):
            if j + 2 < IDX_ROWS:
                if j >= 1:
                    sdescs[j - 1].wait()
                descs[j + 2] = pltpu.async_copy(
                    table.at[srcv.at[p, j + 2]], bufs[(j + 2) % 3],
                    sems[(j + 2) % 3])
            descs[j].wait()
            sdescs[j] = pltpu.async_copy(bufs[j % 3], acc.at[dstv.at[p, j]],
                                         ssems[j % 3], add=True)
            if with_deg:
                pltpu.sync_copy(ones1, dacc.at[dstv.at[p, j]], add=True)
        for j in range(max(IDX_ROWS - 3, 1), IDX_ROWS):
            sdescs[j].wait()

    def idx_drain(p):
        pltpu.make_async_copy(src_i.at[pl.ds(0, IDX_ROWS)], srcv.at[p],
                              semc).wait()
        pltpu.make_async_copy(dst_i.at[pl.ds(0, IDX_ROWS)], dstv.at[p],
                              semd).wait()

    @pl.loop(0, npair)
    def _outer(bb):
        b0 = bb * 2
        idx_load(b0 + 1, 1)
        idx_drain(0)
        process(0)
        idx_load(b0 + 2, 0)
        idx_drain(1)
        process(1)

    idx_drain(0)
    plsc.subcore_barrier()
    base = c * ACC_ROWS + row0
    loads = [None] * nchunk
    outs = [None] * nchunk
    loads[0] = pltpu.async_copy(acc.at[pl.ds(row0, CHUNK)], rowsa, sema)
    for t in range(nchunk):
        bt = bufs[t % 2]
        if t + 1 < nchunk:
            if t >= 1:
                outs[t - 1].wait()
            loads[t + 1] = pltpu.async_copy(
                acc.at[pl.ds(row0 + (t + 1) * CHUNK, CHUNK)],
                bufs[(t + 1) % 2], sems[(t + 1) % 2])
        loads[t].wait()
        outs[t] = pltpu.async_copy(bt, out.at[pl.ds(base + t * CHUNK, CHUNK)],
                                   semc if t % 2 == 0 else semd)
    outs[nchunk - 2].wait()
    outs[nchunk - 1].wait()
    if with_deg:
        pltpu.sync_copy(dacc.at[pl.ds(row0, ROWS_PER_TILE)], dstage)
        pltpu.sync_copy(dstage, deg_out.at[pl.ds(base, ROWS_PER_TILE)])


def _make_seg_kernel(n_edges_pad, with_deg):
    nblk = n_edges_pad // (NC * NS * EDGE_BLK)
    mesh = plsc.VectorSubcoreMesh(core_axis_name="c", subcore_axis_name="s")
    outs = [jax.ShapeDtypeStruct((NC * ACC_ROWS, HID), jnp.float32)]
    scratch = [
        pltpu.VMEM_SHARED((ACC_ROWS, HID), jnp.float32),
    ]
    if with_deg:
        outs.append(jax.ShapeDtypeStruct((NC * ACC_ROWS,), jnp.float32))
        scratch.append(pltpu.VMEM_SHARED((ACC_ROWS,), jnp.float32))
    scratch += [
        pltpu.VMEM((2, IDX_ROWS, CHUNK), jnp.int32),
        pltpu.VMEM((2, IDX_ROWS, CHUNK), jnp.int32),
        pltpu.VMEM((CHUNK, HID), jnp.float32),
        pltpu.VMEM((CHUNK, HID), jnp.float32),
        pltpu.VMEM((CHUNK, HID), jnp.float32),
    ]
    if with_deg:
        scratch += [
            pltpu.VMEM((CHUNK,), jnp.float32),
            pltpu.VMEM((ROWS_PER_TILE,), jnp.float32),
        ]
    scratch += [pltpu.SemaphoreType.DMA] * 8
    return pl.kernel(
        functools.partial(_seg_body, nblk, with_deg),
        out_type=tuple(outs) if with_deg else outs[0],
        mesh=mesh,
        scratch_types=scratch,
    )


def _stage_a_body(x, wf, ba, wt, bt, w1, b1, wr, br, a, g1, r):
    av = a[0, 0]
    e = _prelu(jnp.dot(x[...], wf[...], preferred_element_type=jnp.float32)
               + ba[...], av)
    h0 = _prelu(jnp.dot(e, wt[...], preferred_element_type=jnp.float32)
                + bt[...], av)
    g1[...] = jnp.dot(h0, w1[...], preferred_element_type=jnp.float32) + b1[...]
    r[...] = jnp.dot(h0, wr[...], preferred_element_type=jnp.float32) + br[...]


def _stage_b_body(sp, dp, w2, b2, g2):
    ssum = sp[0] + sp[1]
    deg = jnp.maximum(dp[0] + dp[1], 1.0)
    h = jnp.maximum(ssum / deg, 0.0)
    g2[...] = jnp.dot(h, w2[...], preferred_element_type=jnp.float32) + b2[...]


def _stage_c_body(sp, dp, r, wp1, bp1, wp2, bp2, out):
    ssum = sp[0] + sp[1]
    deg = jnp.maximum(dp[0] + dp[1], 1.0)
    h2 = jnp.maximum(ssum / deg, 0.0)
    t = h2 + r[...]
    p = jnp.maximum(jnp.dot(t, wp1[...], preferred_element_type=jnp.float32)
                    + bp1[...], 0.0)
    out[...] = jnp.dot(p, wp2[...], preferred_element_type=jnp.float32) + bp2[...]


def _full(shape):
    return pl.BlockSpec(shape, lambda i: tuple(0 for _ in shape))


def kernel(x, edge_index, edge_type, W_des, b_des, W_num, b_num, W_cat, b_cat,
           W_tot, b_tot, a_emb, W1, b1, W2, b2, Wr, br, Wp1, bp1, Wp2, bp2):
    del edge_type
    f32 = jnp.float32
    n = x.shape[0]
    k = x.shape[1]

    num_idx = np.array([4, 6, 7, 8, 10, 11, 12, 13, 14, 15])
    cat_idx = np.array([1, 2, 3, 5, 9, 16, 17, 18, 19, 20])
    ktop = k - 768
    m_num = np.zeros((ktop, 10), np.float32)
    m_num[num_idx, np.arange(10)] = 1.0
    m_cat = np.zeros((ktop, 10), np.float32)
    m_cat[cat_idx, np.arange(10)] = 1.0
    top = jnp.concatenate(
        [jnp.zeros((ktop, 96), f32), jnp.asarray(m_num) @ W_num,
         jnp.asarray(m_cat) @ W_cat], axis=1)
    bot = jnp.concatenate([W_des, jnp.zeros((768, 32), f32)], axis=1)
    wf = jnp.concatenate([top, bot], axis=0)
    ba = jnp.concatenate([b_des, b_num, b_cat]).reshape(1, HID)

    bm = 1000
    grid = (n // bm,)
    row_spec = pl.BlockSpec((bm, HID), lambda i: (i, 0))
    g1, r = pl.pallas_call(
        _stage_a_body,
        grid=grid,
        in_specs=[pl.BlockSpec((bm, k), lambda i: (i, 0)),
                  _full((k, HID)), _full((1, HID)),
                  _full((HID, HID)), _full((1, HID)),
                  _full((HID, HID)), _full((1, HID)),
                  _full((HID, HID)), _full((1, HID)),
                  _full((1, 1))],
        out_specs=[row_spec, row_spec],
        out_shape=[jax.ShapeDtypeStruct((n, HID), f32),
                   jax.ShapeDtypeStruct((n, HID), f32)],
    )(x, wf, ba, W_tot, b_tot.reshape(1, HID), W1, b1.reshape(1, HID),
      Wr, br.reshape(1, HID), a_emb.reshape(1, 1))

    src = edge_index[0]
    dst = edge_index[1]
    n_edges = src.shape[0]
    epad = ((n_edges + NC * NS * EDGE_BLK - 1) // (NC * NS * EDGE_BLK)
            * (NC * NS * EDGE_BLK))
    npad = epad - n_edges
    extra = IDX_ROWS * CHUNK
    pad_i = np.arange(npad + extra, dtype=np.int32)
    pad_src = jnp.asarray(pad_i % N_NODES)
    pad_dst = jnp.asarray(N_NODES + pad_i % (ACC_ROWS - N_NODES))
    src_p = jnp.concatenate([src, pad_src]).reshape(-1, CHUNK)
    dst_p = jnp.concatenate([dst, pad_dst]).reshape(-1, CHUNK)

    z128 = jnp.zeros((CHUNK, HID), f32)

    s1, deg1 = _make_seg_kernel(epad, True)(g1, src_p, dst_p, z128)
    s1 = s1.reshape(NC, ACC_ROWS, HID)
    deg1 = deg1.reshape(NC, ACC_ROWS, 1)

    bm2 = 1024
    grid2 = (ACC_ROWS // bm2,)
    g2 = pl.pallas_call(
        _stage_b_body,
        grid=grid2,
        in_specs=[pl.BlockSpec((NC, bm2, HID), lambda i: (0, i, 0)),
                  pl.BlockSpec((NC, bm2, 1), lambda i: (0, i, 0)),
                  _full((HID, HID)), _full((1, HID))],
        out_specs=pl.BlockSpec((bm2, HID), lambda i: (i, 0)),
        out_shape=jax.ShapeDtypeStruct((ACC_ROWS, HID), f32),
    )(s1, deg1, W2, b2.reshape(1, HID))

    s2 = _make_seg_kernel(epad, False)(g2, src_p, dst_p, z128)
    s2 = s2.reshape(NC, ACC_ROWS, HID)

    wp2 = jnp.zeros((HID, HID), f32).at[:, :2].set(Wp2)
    bp2p = jnp.zeros((1, HID), f32).at[0, :2].set(bp2)
    out = pl.pallas_call(
        _stage_c_body,
        grid=grid,
        in_specs=[pl.BlockSpec((NC, bm, HID), lambda i: (0, i, 0)),
                  pl.BlockSpec((NC, bm, 1), lambda i: (0, i, 0)),
                  row_spec,
                  _full((HID, HID)), _full((1, HID)),
                  _full((HID, HID)), _full((1, HID))],
        out_specs=row_spec,
        out_shape=jax.ShapeDtypeStruct((n, HID), f32),
    )(s2, deg1, r, Wp1, bp1.reshape(1, HID), wp2, bp2p)

    return out[:, :2]
